# baseline (speedup 1.0000x reference)
import functools

import jax
import jax.numpy as jnp
from jax import lax
from jax.experimental import pallas as pl
from jax.experimental.pallas import tpu as pltpu
from jax.experimental.pallas import tpu_sc as plsc

N = 10000
D = 128
E = 320000
NC = 2
NS = 16
NW = NC * NS
K = 64
NITER = 160
EPAD = NW * NITER * K
NPAD = 10240
STRIPE = NPAD // NS
PH = NITER // 2
NB = 10
RB = N // NB
EPS = 1e-5

_mesh = plsc.VectorSubcoreMesh(core_axis_name="c", subcore_axis_name="s",
                               num_cores=NC, num_subcores=NS)


def _sc_degree_body(dst_hbm, ones_hbm, zeros_hbm, degp_out, idx_v, ones_v,
                    deg_sp):
    c = lax.axis_index("c")
    s = lax.axis_index("s")
    wid = c * NS + s
    pltpu.sync_copy(zeros_hbm, deg_sp.at[pl.ds(s * STRIPE, STRIPE)])
    pltpu.sync_copy(dst_hbm.at[wid], idx_v)
    pltpu.sync_copy(ones_hbm, ones_v)
    plsc.subcore_barrier()

    def body(j, carry):
        pltpu.sync_copy(ones_v, deg_sp.at[idx_v.at[j]], add=True)
        return carry

    lax.fori_loop(0, NITER, body, 0)
    plsc.subcore_barrier()
    pltpu.sync_copy(deg_sp.at[pl.ds(s * STRIPE, STRIPE)],
                    degp_out.at[c, pl.ds(s * STRIPE, STRIPE)])


_sc_degree = functools.partial(
    pl.kernel,
    out_type=jax.ShapeDtypeStruct((NC, NPAD, D), jnp.float32),
    mesh=_mesh,
    scratch_types=[
        pltpu.VMEM((NITER, K), jnp.int32),
        pltpu.VMEM((K, D), jnp.float32),
        pltpu.VMEM_SHARED((NPAD, D), jnp.float32),
    ],
)(_sc_degree_body)


NBUF = 3


def _sc_aggregate_body(hs_hbm, src_hbm, dst_hbm, zeros_hbm, aggp_out,
                       src_v, dst_v, rows_v, gsem, ssem, agg_sp):
    c = lax.axis_index("c")
    s = lax.axis_index("s")
    wid = c * NS + s

    def run_phase(base, cnt, zero_and_barrier):
        pltpu.sync_copy(src_hbm.at[wid, pl.ds(base, cnt)],
                        src_v.at[pl.ds(0, cnt)])
        pltpu.sync_copy(dst_hbm.at[wid, pl.ds(base, cnt)],
                        dst_v.at[pl.ds(0, cnt)])
        pltpu.async_copy(hs_hbm.at[src_v.at[0]], rows_v.at[0], gsem.at[0])
        pltpu.async_copy(hs_hbm.at[src_v.at[1]], rows_v.at[1], gsem.at[1])
        if zero_and_barrier:
            pltpu.sync_copy(zeros_hbm, agg_sp.at[pl.ds(s * STRIPE, STRIPE)])
            plsc.subcore_barrier()

        def body(t, carry):
            b = lax.rem(t, NBUF)
            pltpu.make_async_copy(hs_hbm.at[src_v.at[t]], rows_v.at[b],
                                  gsem.at[b]).wait()
            pltpu.async_copy(rows_v.at[b], agg_sp.at[dst_v.at[t]],
                             ssem.at[b], add=True)

            @pl.when(t < cnt - 2)
            def _launch_next():
                t2 = t + 2
                b2 = lax.rem(t2, NBUF)

                @pl.when(t2 >= NBUF)
                def _drain_prev():
                    pltpu.make_async_copy(rows_v.at[b2],
                                          agg_sp.at[dst_v.at[0]],
                                          ssem.at[b2]).wait()

                pltpu.async_copy(hs_hbm.at[src_v.at[t2]], rows_v.at[b2],
                                 gsem.at[b2])

            return carry

        lax.fori_loop(0, cnt, body, 0)
        for b in range(NBUF):
            pltpu.make_async_copy(rows_v.at[b], agg_sp.at[dst_v.at[0]],
                                  ssem.at[b]).wait()

    run_phase(0, PH, True)
    run_phase(PH, PH, False)
    plsc.subcore_barrier()
    pltpu.sync_copy(agg_sp.at[pl.ds(s * STRIPE, STRIPE)],
                    aggp_out.at[c, pl.ds(s * STRIPE, STRIPE)])


_sc_aggregate = functools.partial(
    pl.kernel,
    out_type=jax.ShapeDtypeStruct((NC, NPAD, D), jnp.float32),
    mesh=_mesh,
    scratch_types=[
        pltpu.VMEM((PH, K), jnp.int32),
        pltpu.VMEM((PH, K), jnp.int32),
        pltpu.VMEM((NBUF, K, D), jnp.float32),
        pltpu.SemaphoreType.DMA((NBUF,)),
        pltpu.SemaphoreType.DMA((NBUF,)),
        pltpu.VMEM_SHARED((NPAD, D), jnp.float32),
    ],
)(_sc_aggregate_body)


def _mm_body(x_ref, w_ref, o_ref):
    o_ref[...] = jnp.dot(x_ref[...], w_ref[...],
                         preferred_element_type=jnp.float32)


def _tc_matmul(x, W):
    return pl.pallas_call(
        _mm_body,
        grid=(NB,),
        in_specs=[
            pl.BlockSpec((RB, D), lambda i: (i, 0)),
            pl.BlockSpec((D, D), lambda i: (0, 0)),
        ],
        out_specs=pl.BlockSpec((RB, D), lambda i: (i, 0)),
        out_shape=jax.ShapeDtypeStruct((N, D), jnp.float32),
    )(x, W)


def _scale_body(h_ref, degp_ref, o_ref):
    deg = 1.0 + degp_ref[0] + degp_ref[1]
    o_ref[...] = h_ref[...] * lax.rsqrt(deg)


def _tc_scale(h, degp):
    return pl.pallas_call(
        _scale_body,
        grid=(NB,),
        in_specs=[
            pl.BlockSpec((RB, D), lambda i: (i, 0)),
            pl.BlockSpec((NC, RB, D), lambda i: (0, i, 0)),
        ],
        out_specs=pl.BlockSpec((RB, D), lambda i: (i, 0)),
        out_shape=jax.ShapeDtypeStruct((N, D), jnp.float32),
    )(h, degp)


def _final_body(aggp_ref, hs_ref, degp_ref, b_ref, lnw_ref, lnb_ref, a_ref,
                o_ref, t_vmem, acc):
    p = pl.program_id(0)
    i = pl.program_id(1)

    @pl.when(p == 0)
    def _phase0():
        @pl.when(i == 0)
        def _init():
            acc[0] = 0.0
            acc[1] = 0.0

        deg = 1.0 + degp_ref[0] + degp_ref[1]
        ag = aggp_ref[0] + aggp_ref[1] + hs_ref[...]
        t = ag * lax.rsqrt(deg) + b_ref[...]
        t_vmem[pl.ds(i * RB, RB), :] = t
        acc[0] += jnp.sum(t)
        acc[1] += jnp.sum(t * t)

    @pl.when(p == 1)
    def _phase1():
        inv_n = 1.0 / (N * D)
        m = acc[0] * inv_n
        var = acc[1] * inv_n - m * m
        std = jnp.sqrt(var)
        t = t_vmem[pl.ds(i * RB, RB), :]
        o = (t - m) / (std + EPS) * lnw_ref[...] + lnb_ref[...]
        a = a_ref[0, 0]
        o_ref[...] = jnp.where(o >= 0.0, o, a * o)


def _tc_finalize(aggp, hs, degp, b, ln_w, ln_b, prelu_a):
    return pl.pallas_call(
        _final_body,
        grid=(2, NB),
        in_specs=[
            pl.BlockSpec((NC, RB, D),
                         lambda p, i: (0, jnp.where(p == 0, i, 0), 0)),
            pl.BlockSpec((RB, D), lambda p, i: (jnp.where(p == 0, i, 0), 0)),
            pl.BlockSpec((NC, RB, D),
                         lambda p, i: (0, jnp.where(p == 0, i, 0), 0)),
            pl.BlockSpec((1, D), lambda p, i: (0, 0)),
            pl.BlockSpec((1, D), lambda p, i: (0, 0)),
            pl.BlockSpec((1, D), lambda p, i: (0, 0)),
            pl.BlockSpec((1, 1), lambda p, i: (0, 0)),
        ],
        out_specs=pl.BlockSpec((RB, D),
                               lambda p, i: (jnp.where(p == 0, 0, i), 0)),
        out_shape=jax.ShapeDtypeStruct((N, D), jnp.float32),
        scratch_shapes=[
            pltpu.VMEM((N, D), jnp.float32),
            pltpu.SMEM((2,), jnp.float32),
        ],
    )(aggp, hs, degp, b, ln_w, ln_b, prelu_a)


def kernel(x, edge_index, W, b, ln_w, ln_b, prelu_a):
    pad = EPAD - E
    src = jnp.concatenate(
        [edge_index[0], jnp.zeros((pad,), jnp.int32)]).reshape(NW, NITER, K)
    dst = jnp.concatenate(
        [edge_index[1], jnp.full((pad,), N, jnp.int32)]).reshape(NW, NITER, K)

    onesd = jnp.ones((K, D), jnp.float32)
    zerosd = jnp.zeros((STRIPE, D), jnp.float32)

    degp = _sc_degree(dst, onesd, zerosd)
    h = _tc_matmul(x, W)
    hs = _tc_scale(h, degp)
    aggp = _sc_aggregate(hs, src, dst, zerosd)

    b2 = b.reshape(1, D)
    lnw2 = ln_w.reshape(1, D)
    lnb2 = ln_b.reshape(1, D)
    a2 = prelu_a.reshape(1, 1)
    return _tc_finalize(aggp, hs, degp, b2, lnw2, lnb2, a2)


# speedup vs baseline: 1.0449x; 1.0449x over previous
import functools

import jax
import jax.numpy as jnp
from jax import lax
from jax.experimental import pallas as pl
from jax.experimental.pallas import tpu as pltpu
from jax.experimental.pallas import tpu_sc as plsc

N = 10000
D = 128
E = 320000
NC = 2
NS = 16
NW = NC * NS
K = 80
NITER = 128
EPAD = NW * NITER * K
NPAD = 10240
STRIPE = NPAD // NS
PH = NITER // 2
NB = 10
RB = N // NB
EPS = 1e-5

_mesh = plsc.VectorSubcoreMesh(core_axis_name="c", subcore_axis_name="s",
                               num_cores=NC, num_subcores=NS)


def _sc_degree_body(dst_hbm, ones_hbm, zeros_hbm, degp_out, idx_v, ones_v,
                    deg_sp):
    c = lax.axis_index("c")
    s = lax.axis_index("s")
    wid = c * NS + s
    pltpu.sync_copy(zeros_hbm, deg_sp.at[pl.ds(s * STRIPE, STRIPE)])
    pltpu.sync_copy(dst_hbm.at[wid], idx_v)
    pltpu.sync_copy(ones_hbm, ones_v)
    plsc.subcore_barrier()

    def body(j, carry):
        pltpu.sync_copy(ones_v, deg_sp.at[idx_v.at[j]], add=True)
        return carry

    lax.fori_loop(0, NITER, body, 0)
    plsc.subcore_barrier()
    pltpu.sync_copy(deg_sp.at[pl.ds(s * STRIPE, STRIPE)],
                    degp_out.at[c, pl.ds(s * STRIPE, STRIPE)])


_sc_degree = functools.partial(
    pl.kernel,
    out_type=jax.ShapeDtypeStruct((NC, NPAD, D), jnp.float32),
    mesh=_mesh,
    scratch_types=[
        pltpu.VMEM((NITER, K), jnp.int32),
        pltpu.VMEM((K, D), jnp.float32),
        pltpu.VMEM_SHARED((NPAD, D), jnp.float32),
    ],
)(_sc_degree_body)


NBUF = 2


def _sc_aggregate_body(hs_hbm, src_hbm, dst_hbm, zeros_hbm, aggp_out,
                       src_v, dst_v, rows_v, gsem, agg_sp):
    c = lax.axis_index("c")
    s = lax.axis_index("s")
    wid = c * NS + s

    def run_phase(base, cnt, zero_and_barrier):
        pltpu.sync_copy(src_hbm.at[wid, pl.ds(base, cnt)],
                        src_v.at[pl.ds(0, cnt)])
        pltpu.sync_copy(dst_hbm.at[wid, pl.ds(base, cnt)],
                        dst_v.at[pl.ds(0, cnt)])
        pltpu.async_copy(hs_hbm.at[src_v.at[0]], rows_v.at[0], gsem.at[0])
        if zero_and_barrier:
            pltpu.sync_copy(zeros_hbm, agg_sp.at[pl.ds(s * STRIPE, STRIPE)])
            plsc.subcore_barrier()

        def body(t, carry):
            b = lax.rem(t, NBUF)
            nb = 1 - b

            @pl.when(t + 1 < cnt)
            def _prefetch():
                pltpu.async_copy(hs_hbm.at[src_v.at[t + 1]], rows_v.at[nb],
                                 gsem.at[nb])

            pltpu.make_async_copy(hs_hbm.at[src_v.at[t]], rows_v.at[b],
                                  gsem.at[b]).wait()
            pltpu.sync_copy(rows_v.at[b], agg_sp.at[dst_v.at[t]], add=True)
            return carry

        lax.fori_loop(0, cnt, body, 0)

    run_phase(0, PH, True)
    run_phase(PH, PH, False)
    plsc.subcore_barrier()
    pltpu.sync_copy(agg_sp.at[pl.ds(s * STRIPE, STRIPE)],
                    aggp_out.at[c, pl.ds(s * STRIPE, STRIPE)])


_sc_aggregate = functools.partial(
    pl.kernel,
    out_type=jax.ShapeDtypeStruct((NC, NPAD, D), jnp.float32),
    mesh=_mesh,
    scratch_types=[
        pltpu.VMEM((PH, K), jnp.int32),
        pltpu.VMEM((PH, K), jnp.int32),
        pltpu.VMEM((NBUF, K, D), jnp.float32),
        pltpu.SemaphoreType.DMA((NBUF,)),
        pltpu.VMEM_SHARED((NPAD, D), jnp.float32),
    ],
)(_sc_aggregate_body)


def _mm_body(x_ref, w_ref, o_ref):
    o_ref[...] = jnp.dot(x_ref[...], w_ref[...],
                         preferred_element_type=jnp.float32)


def _tc_matmul(x, W):
    return pl.pallas_call(
        _mm_body,
        grid=(NB,),
        in_specs=[
            pl.BlockSpec((RB, D), lambda i: (i, 0)),
            pl.BlockSpec((D, D), lambda i: (0, 0)),
        ],
        out_specs=pl.BlockSpec((RB, D), lambda i: (i, 0)),
        out_shape=jax.ShapeDtypeStruct((N, D), jnp.float32),
    )(x, W)


def _scale_body(h_ref, degp_ref, o_ref):
    deg = 1.0 + degp_ref[0] + degp_ref[1]
    o_ref[...] = h_ref[...] * lax.rsqrt(deg)


def _tc_scale(h, degp):
    return pl.pallas_call(
        _scale_body,
        grid=(NB,),
        in_specs=[
            pl.BlockSpec((RB, D), lambda i: (i, 0)),
            pl.BlockSpec((NC, RB, D), lambda i: (0, i, 0)),
        ],
        out_specs=pl.BlockSpec((RB, D), lambda i: (i, 0)),
        out_shape=jax.ShapeDtypeStruct((N, D), jnp.float32),
    )(h, degp)


def _final_body(aggp_ref, hs_ref, degp_ref, b_ref, lnw_ref, lnb_ref, a_ref,
                o_ref, t_vmem, acc):
    p = pl.program_id(0)
    i = pl.program_id(1)

    @pl.when(p == 0)
    def _phase0():
        @pl.when(i == 0)
        def _init():
            acc[0] = 0.0
            acc[1] = 0.0

        deg = 1.0 + degp_ref[0] + degp_ref[1]
        ag = aggp_ref[0] + aggp_ref[1] + hs_ref[...]
        t = ag * lax.rsqrt(deg) + b_ref[...]
        t_vmem[pl.ds(i * RB, RB), :] = t
        acc[0] += jnp.sum(t)
        acc[1] += jnp.sum(t * t)

    @pl.when(p == 1)
    def _phase1():
        inv_n = 1.0 / (N * D)
        m = acc[0] * inv_n
        var = acc[1] * inv_n - m * m
        std = jnp.sqrt(var)
        t = t_vmem[pl.ds(i * RB, RB), :]
        o = (t - m) / (std + EPS) * lnw_ref[...] + lnb_ref[...]
        a = a_ref[0, 0]
        o_ref[...] = jnp.where(o >= 0.0, o, a * o)


def _tc_finalize(aggp, hs, degp, b, ln_w, ln_b, prelu_a):
    return pl.pallas_call(
        _final_body,
        grid=(2, NB),
        in_specs=[
            pl.BlockSpec((NC, RB, D),
                         lambda p, i: (0, jnp.where(p == 0, i, 0), 0)),
            pl.BlockSpec((RB, D), lambda p, i: (jnp.where(p == 0, i, 0), 0)),
            pl.BlockSpec((NC, RB, D),
                         lambda p, i: (0, jnp.where(p == 0, i, 0), 0)),
            pl.BlockSpec((1, D), lambda p, i: (0, 0)),
            pl.BlockSpec((1, D), lambda p, i: (0, 0)),
            pl.BlockSpec((1, D), lambda p, i: (0, 0)),
            pl.BlockSpec((1, 1), lambda p, i: (0, 0)),
        ],
        out_specs=pl.BlockSpec((RB, D),
                               lambda p, i: (jnp.where(p == 0, 0, i), 0)),
        out_shape=jax.ShapeDtypeStruct((N, D), jnp.float32),
        scratch_shapes=[
            pltpu.VMEM((N, D), jnp.float32),
            pltpu.SMEM((2,), jnp.float32),
        ],
    )(aggp, hs, degp, b, ln_w, ln_b, prelu_a)


def kernel(x, edge_index, W, b, ln_w, ln_b, prelu_a):
    pad = EPAD - E
    src = jnp.concatenate(
        [edge_index[0], jnp.zeros((pad,), jnp.int32)]).reshape(NW, NITER, K)
    dst = jnp.concatenate(
        [edge_index[1], jnp.full((pad,), N, jnp.int32)]).reshape(NW, NITER, K)

    onesd = jnp.ones((K, D), jnp.float32)
    zerosd = jnp.zeros((STRIPE, D), jnp.float32)

    degp = _sc_degree(dst, onesd, zerosd)
    h = _tc_matmul(x, W)
    hs = _tc_scale(h, degp)
    aggp = _sc_aggregate(hs, src, dst, zerosd)

    b2 = b.reshape(1, D)
    lnw2 = ln_w.reshape(1, D)
    lnb2 = ln_b.reshape(1, D)
    a2 = prelu_a.reshape(1, 1)
    return _tc_finalize(aggp, hs, degp, b2, lnw2, lnb2, a2)


# speedup vs baseline: 2.2802x; 2.1822x over previous
import functools

import jax
import jax.numpy as jnp
from jax import lax
from jax.experimental import pallas as pl
from jax.experimental.pallas import tpu as pltpu
from jax.experimental.pallas import tpu_sc as plsc

N = 10000
D = 128
E = 320000
NC = 2
NS = 16
NW = NC * NS
K = 80
NITER = 128
EPAD = NW * NITER * K
NPAD = 10240
STRIPE = NPAD // NS
PH = NITER // 2
NB = 10
RB = N // NB
EPS = 1e-5

_mesh = plsc.VectorSubcoreMesh(core_axis_name="c", subcore_axis_name="s",
                               num_cores=NC, num_subcores=NS)


def _sc_degree_body(dst_hbm, ones_hbm, zeros_hbm, degp_out, idx_v, ones_v,
                    deg_sp):
    c = lax.axis_index("c")
    s = lax.axis_index("s")
    wid = c * NS + s
    pltpu.sync_copy(zeros_hbm, deg_sp.at[pl.ds(s * STRIPE, STRIPE)])
    pltpu.sync_copy(dst_hbm.at[wid], idx_v)
    pltpu.sync_copy(ones_hbm, ones_v)
    plsc.subcore_barrier()

    def body(j, carry):
        pltpu.sync_copy(ones_v, deg_sp.at[idx_v.at[j]], add=True)
        return carry

    lax.fori_loop(0, NITER, body, 0)
    plsc.subcore_barrier()
    pltpu.sync_copy(deg_sp.at[pl.ds(s * STRIPE, STRIPE)],
                    degp_out.at[c, pl.ds(s * STRIPE, STRIPE)])


_sc_degree = functools.partial(
    pl.kernel,
    out_type=jax.ShapeDtypeStruct((NC, NPAD, D), jnp.float32),
    mesh=_mesh,
    scratch_types=[
        pltpu.VMEM((NITER, K), jnp.int32),
        pltpu.VMEM((K, D), jnp.float32),
        pltpu.VMEM_SHARED((NPAD, D), jnp.float32),
    ],
)(_sc_degree_body)


NBUF = 2


def _sc_aggregate_body(hs_hbm, src_hbm, dst_hbm, zeros_hbm, aggp_out,
                       src_v, dst_v, rows_v, gsem, agg_sp):
    c = lax.axis_index("c")
    s = lax.axis_index("s")
    wid = c * NS + s

    def run_phase(base, cnt, zero_and_barrier):
        pltpu.sync_copy(src_hbm.at[wid, pl.ds(base, cnt)],
                        src_v.at[pl.ds(0, cnt)])
        pltpu.sync_copy(dst_hbm.at[wid, pl.ds(base, cnt)],
                        dst_v.at[pl.ds(0, cnt)])
        pltpu.async_copy(hs_hbm.at[src_v.at[0]], rows_v.at[0], gsem.at[0])
        if zero_and_barrier:
            pltpu.sync_copy(zeros_hbm, agg_sp.at[pl.ds(s * STRIPE, STRIPE)])
            plsc.subcore_barrier()

        def body(t, carry):
            b = lax.rem(t, NBUF)
            nb = 1 - b

            @pl.when(t + 1 < cnt)
            def _prefetch():
                pltpu.async_copy(hs_hbm.at[src_v.at[t + 1]], rows_v.at[nb],
                                 gsem.at[nb])

            pltpu.make_async_copy(hs_hbm.at[src_v.at[t]], rows_v.at[b],
                                  gsem.at[b]).wait()
            pltpu.sync_copy(rows_v.at[b], agg_sp.at[dst_v.at[t]], add=True)
            return carry

        lax.fori_loop(0, cnt, body, 0)

    run_phase(0, PH, True)
    run_phase(PH, PH, False)
    plsc.subcore_barrier()
    pltpu.sync_copy(agg_sp.at[pl.ds(s * STRIPE, STRIPE)],
                    aggp_out.at[c, pl.ds(s * STRIPE, STRIPE)])


_sc_aggregate = functools.partial(
    pl.kernel,
    out_type=jax.ShapeDtypeStruct((NC, NPAD, D), jnp.float32),
    mesh=_mesh,
    scratch_types=[
        pltpu.VMEM((PH, K), jnp.int32),
        pltpu.VMEM((PH, K), jnp.int32),
        pltpu.VMEM((NBUF, K, D), jnp.float32),
        pltpu.SemaphoreType.DMA((NBUF,)),
        pltpu.VMEM_SHARED((NPAD, D), jnp.float32),
    ],
)(_sc_aggregate_body)


def _mm_body(x_ref, w_ref, o_ref):
    o_ref[...] = jnp.dot(x_ref[...], w_ref[...],
                         preferred_element_type=jnp.float32)


def _tc_matmul(x, W):
    return pl.pallas_call(
        _mm_body,
        grid=(NB,),
        in_specs=[
            pl.BlockSpec((RB, D), lambda i: (i, 0)),
            pl.BlockSpec((D, D), lambda i: (0, 0)),
        ],
        out_specs=pl.BlockSpec((RB, D), lambda i: (i, 0)),
        out_shape=jax.ShapeDtypeStruct((N, D), jnp.float32),
    )(x, W)


def _scale_body(h_ref, degp_ref, o_ref):
    deg = 1.0 + degp_ref[0] + degp_ref[1]
    o_ref[...] = h_ref[...] * lax.rsqrt(deg)


def _tc_scale(h, degp):
    return pl.pallas_call(
        _scale_body,
        grid=(NB,),
        in_specs=[
            pl.BlockSpec((RB, D), lambda i: (i, 0)),
            pl.BlockSpec((NC, RB, D), lambda i: (0, i, 0)),
        ],
        out_specs=pl.BlockSpec((RB, D), lambda i: (i, 0)),
        out_shape=jax.ShapeDtypeStruct((N, D), jnp.float32),
    )(h, degp)


def _final_body(aggp_ref, hs_ref, degp_ref, b_ref, lnw_ref, lnb_ref, a_ref,
                o_ref, t_vmem, acc):
    p = pl.program_id(0)
    i = pl.program_id(1)

    @pl.when(p == 0)
    def _phase0():
        @pl.when(i == 0)
        def _init():
            acc[0] = 0.0
            acc[1] = 0.0

        deg = 1.0 + degp_ref[0] + degp_ref[1]
        ag = aggp_ref[0] + aggp_ref[1] + hs_ref[...]
        t = ag * lax.rsqrt(deg) + b_ref[...]
        t_vmem[pl.ds(i * RB, RB), :] = t
        acc[0] += jnp.sum(t)
        acc[1] += jnp.sum(t * t)

    @pl.when(p == 1)
    def _phase1():
        inv_n = 1.0 / (N * D)
        m = acc[0] * inv_n
        var = acc[1] * inv_n - m * m
        std = jnp.sqrt(var)
        t = t_vmem[pl.ds(i * RB, RB), :]
        o = (t - m) / (std + EPS) * lnw_ref[...] + lnb_ref[...]
        a = a_ref[0, 0]
        o_ref[...] = jnp.where(o >= 0.0, o, a * o)


def _tc_finalize(aggp, hs, degp, b, ln_w, ln_b, prelu_a):
    return pl.pallas_call(
        _final_body,
        grid=(2, NB),
        in_specs=[
            pl.BlockSpec((NC, RB, D),
                         lambda p, i: (0, jnp.where(p == 0, i, 0), 0)),
            pl.BlockSpec((RB, D), lambda p, i: (jnp.where(p == 0, i, 0), 0)),
            pl.BlockSpec((NC, RB, D),
                         lambda p, i: (0, jnp.where(p == 0, i, 0), 0)),
            pl.BlockSpec((1, D), lambda p, i: (0, 0)),
            pl.BlockSpec((1, D), lambda p, i: (0, 0)),
            pl.BlockSpec((1, D), lambda p, i: (0, 0)),
            pl.BlockSpec((1, 1), lambda p, i: (0, 0)),
        ],
        out_specs=pl.BlockSpec((RB, D),
                               lambda p, i: (jnp.where(p == 0, 0, i), 0)),
        out_shape=jax.ShapeDtypeStruct((N, D), jnp.float32),
        scratch_shapes=[
            pltpu.VMEM((N, D), jnp.float32),
            pltpu.SMEM((2,), jnp.float32),
        ],
    )(aggp, hs, degp, b, ln_w, ln_b, prelu_a)


def kernel(x, edge_index, W, b, ln_w, ln_b, prelu_a):
    pad = EPAD - E
    pad_ar = jnp.arange(pad, dtype=jnp.int32)
    src = jnp.concatenate(
        [edge_index[0], pad_ar % N]).reshape(NW, NITER, K)
    dst = jnp.concatenate(
        [edge_index[1], N + pad_ar % (NPAD - N)]).reshape(NW, NITER, K)

    onesd = jnp.ones((K, D), jnp.float32)
    zerosd = jnp.zeros((STRIPE, D), jnp.float32)

    degp = _sc_degree(dst, onesd, zerosd)
    h = _tc_matmul(x, W)
    hs = _tc_scale(h, degp)
    aggp = _sc_aggregate(hs, src, dst, zerosd)

    b2 = b.reshape(1, D)
    lnw2 = ln_w.reshape(1, D)
    lnb2 = ln_b.reshape(1, D)
    a2 = prelu_a.reshape(1, 1)
    return _tc_finalize(aggp, hs, degp, b2, lnw2, lnb2, a2)


# speedup vs baseline: 2.4175x; 1.0602x over previous
import functools

import jax
import jax.numpy as jnp
from jax import lax
from jax.experimental import pallas as pl
from jax.experimental.pallas import tpu as pltpu
from jax.experimental.pallas import tpu_sc as plsc

N = 10000
D = 128
E = 320000
NC = 2
NS = 16
NW = NC * NS
K = 64
NITER = 160
EPAD = NW * NITER * K
NPAD = 10240
STRIPE = NPAD // NS
PH = NITER // 2
NB = 10
RB = N // NB
EPS = 1e-5

_mesh = plsc.VectorSubcoreMesh(core_axis_name="c", subcore_axis_name="s",
                               num_cores=NC, num_subcores=NS)


def _sc_degree_body(dst_hbm, ones_hbm, zeros_hbm, degp_out, idx_v, ones_v,
                    deg_sp):
    c = lax.axis_index("c")
    s = lax.axis_index("s")
    wid = c * NS + s
    pltpu.sync_copy(zeros_hbm, deg_sp.at[pl.ds(s * STRIPE, STRIPE)])
    pltpu.sync_copy(dst_hbm.at[wid], idx_v)
    pltpu.sync_copy(ones_hbm, ones_v)
    plsc.subcore_barrier()

    def body(j, carry):
        pltpu.sync_copy(ones_v, deg_sp.at[idx_v.at[j]], add=True)
        return carry

    lax.fori_loop(0, NITER, body, 0)
    plsc.subcore_barrier()
    pltpu.sync_copy(deg_sp.at[pl.ds(s * STRIPE, STRIPE)],
                    degp_out.at[c, pl.ds(s * STRIPE, STRIPE)])


_sc_degree = functools.partial(
    pl.kernel,
    out_type=jax.ShapeDtypeStruct((NC, NPAD, D), jnp.float32),
    mesh=_mesh,
    scratch_types=[
        pltpu.VMEM((NITER, K), jnp.int32),
        pltpu.VMEM((K, D), jnp.float32),
        pltpu.VMEM_SHARED((NPAD, D), jnp.float32),
    ],
)(_sc_degree_body)


NBUF = 3


def _sc_aggregate_body(hs_hbm, src_hbm, dst_hbm, zeros_hbm, aggp_out,
                       src_v, dst_v, rows_v, gsem, agg_sp):
    c = lax.axis_index("c")
    s = lax.axis_index("s")
    wid = c * NS + s

    def run_phase(base, cnt, zero_and_barrier):
        pltpu.sync_copy(src_hbm.at[wid, pl.ds(base, cnt)],
                        src_v.at[pl.ds(0, cnt)])
        pltpu.sync_copy(dst_hbm.at[wid, pl.ds(base, cnt)],
                        dst_v.at[pl.ds(0, cnt)])
        pltpu.async_copy(hs_hbm.at[src_v.at[0]], rows_v.at[0], gsem.at[0])
        pltpu.async_copy(hs_hbm.at[src_v.at[1]], rows_v.at[1], gsem.at[1])
        if zero_and_barrier:
            pltpu.sync_copy(zeros_hbm, agg_sp.at[pl.ds(s * STRIPE, STRIPE)])
            plsc.subcore_barrier()

        def body(t, carry):
            b = lax.rem(t, NBUF)
            nb = lax.rem(t + 2, NBUF)

            @pl.when(t + 2 < cnt)
            def _prefetch():
                pltpu.async_copy(hs_hbm.at[src_v.at[t + 2]], rows_v.at[nb],
                                 gsem.at[nb])

            pltpu.make_async_copy(hs_hbm.at[src_v.at[t]], rows_v.at[b],
                                  gsem.at[b]).wait()
            pltpu.sync_copy(rows_v.at[b], agg_sp.at[dst_v.at[t]], add=True)
            return carry

        lax.fori_loop(0, cnt, body, 0)

    run_phase(0, PH, True)
    run_phase(PH, PH, False)
    plsc.subcore_barrier()
    pltpu.sync_copy(agg_sp.at[pl.ds(s * STRIPE, STRIPE)],
                    aggp_out.at[c, pl.ds(s * STRIPE, STRIPE)])


_sc_aggregate = functools.partial(
    pl.kernel,
    out_type=jax.ShapeDtypeStruct((NC, NPAD, D), jnp.float32),
    mesh=_mesh,
    scratch_types=[
        pltpu.VMEM((PH, K), jnp.int32),
        pltpu.VMEM((PH, K), jnp.int32),
        pltpu.VMEM((NBUF, K, D), jnp.float32),
        pltpu.SemaphoreType.DMA((NBUF,)),
        pltpu.VMEM_SHARED((NPAD, D), jnp.float32),
    ],
)(_sc_aggregate_body)


def _mm_body(x_ref, w_ref, o_ref):
    o_ref[...] = jnp.dot(x_ref[...], w_ref[...],
                         preferred_element_type=jnp.float32)


def _tc_matmul(x, W):
    return pl.pallas_call(
        _mm_body,
        grid=(NB,),
        in_specs=[
            pl.BlockSpec((RB, D), lambda i: (i, 0)),
            pl.BlockSpec((D, D), lambda i: (0, 0)),
        ],
        out_specs=pl.BlockSpec((RB, D), lambda i: (i, 0)),
        out_shape=jax.ShapeDtypeStruct((N, D), jnp.float32),
    )(x, W)


def _scale_body(h_ref, degp_ref, o_ref):
    deg = 1.0 + degp_ref[0] + degp_ref[1]
    o_ref[...] = h_ref[...] * lax.rsqrt(deg)


def _tc_scale(h, degp):
    return pl.pallas_call(
        _scale_body,
        grid=(NB,),
        in_specs=[
            pl.BlockSpec((RB, D), lambda i: (i, 0)),
            pl.BlockSpec((NC, RB, D), lambda i: (0, i, 0)),
        ],
        out_specs=pl.BlockSpec((RB, D), lambda i: (i, 0)),
        out_shape=jax.ShapeDtypeStruct((N, D), jnp.float32),
    )(h, degp)


def _final_body(aggp_ref, hs_ref, degp_ref, b_ref, lnw_ref, lnb_ref, a_ref,
                o_ref, t_vmem, acc):
    p = pl.program_id(0)
    i = pl.program_id(1)

    @pl.when(p == 0)
    def _phase0():
        @pl.when(i == 0)
        def _init():
            acc[0] = 0.0
            acc[1] = 0.0

        deg = 1.0 + degp_ref[0] + degp_ref[1]
        ag = aggp_ref[0] + aggp_ref[1] + hs_ref[...]
        t = ag * lax.rsqrt(deg) + b_ref[...]
        t_vmem[pl.ds(i * RB, RB), :] = t
        acc[0] += jnp.sum(t)
        acc[1] += jnp.sum(t * t)

    @pl.when(p == 1)
    def _phase1():
        inv_n = 1.0 / (N * D)
        m = acc[0] * inv_n
        var = acc[1] * inv_n - m * m
        std = jnp.sqrt(var)
        t = t_vmem[pl.ds(i * RB, RB), :]
        o = (t - m) / (std + EPS) * lnw_ref[...] + lnb_ref[...]
        a = a_ref[0, 0]
        o_ref[...] = jnp.where(o >= 0.0, o, a * o)


def _tc_finalize(aggp, hs, degp, b, ln_w, ln_b, prelu_a):
    return pl.pallas_call(
        _final_body,
        grid=(2, NB),
        in_specs=[
            pl.BlockSpec((NC, RB, D),
                         lambda p, i: (0, jnp.where(p == 0, i, 0), 0)),
            pl.BlockSpec((RB, D), lambda p, i: (jnp.where(p == 0, i, 0), 0)),
            pl.BlockSpec((NC, RB, D),
                         lambda p, i: (0, jnp.where(p == 0, i, 0), 0)),
            pl.BlockSpec((1, D), lambda p, i: (0, 0)),
            pl.BlockSpec((1, D), lambda p, i: (0, 0)),
            pl.BlockSpec((1, D), lambda p, i: (0, 0)),
            pl.BlockSpec((1, 1), lambda p, i: (0, 0)),
        ],
        out_specs=pl.BlockSpec((RB, D),
                               lambda p, i: (jnp.where(p == 0, 0, i), 0)),
        out_shape=jax.ShapeDtypeStruct((N, D), jnp.float32),
        scratch_shapes=[
            pltpu.VMEM((N, D), jnp.float32),
            pltpu.SMEM((2,), jnp.float32),
        ],
    )(aggp, hs, degp, b, ln_w, ln_b, prelu_a)


def kernel(x, edge_index, W, b, ln_w, ln_b, prelu_a):
    pad = EPAD - E
    pad_ar = jnp.arange(pad, dtype=jnp.int32)
    src = jnp.concatenate(
        [edge_index[0], pad_ar % N]).reshape(NW, NITER, K)
    dst = jnp.concatenate(
        [edge_index[1], N + pad_ar % (NPAD - N)]).reshape(NW, NITER, K)

    onesd = jnp.ones((K, D), jnp.float32)
    zerosd = jnp.zeros((STRIPE, D), jnp.float32)

    degp = _sc_degree(dst, onesd, zerosd)
    h = _tc_matmul(x, W)
    hs = _tc_scale(h, degp)
    aggp = _sc_aggregate(hs, src, dst, zerosd)

    b2 = b.reshape(1, D)
    lnw2 = ln_w.reshape(1, D)
    lnb2 = ln_b.reshape(1, D)
    a2 = prelu_a.reshape(1, 1)
    return _tc_finalize(aggp, hs, degp, b2, lnw2, lnb2, a2)


# speedup vs baseline: 2.4505x; 1.0137x over previous
import functools

import jax
import jax.numpy as jnp
from jax import lax
from jax.experimental import pallas as pl
from jax.experimental.pallas import tpu as pltpu
from jax.experimental.pallas import tpu_sc as plsc

N = 10000
D = 128
E = 320000
NC = 2
NS = 16
NW = NC * NS
K = 64
NITER = 160
EPAD = NW * NITER * K
NPAD = 10240
STRIPE = NPAD // NS
PH = NITER // 2
NB = 10
RB = N // NB
EPS = 1e-5

_mesh = plsc.VectorSubcoreMesh(core_axis_name="c", subcore_axis_name="s",
                               num_cores=NC, num_subcores=NS)


def _sc_degree_body(dst_hbm, ones_hbm, zeros_hbm, degp_out, idx_v, ones_v,
                    deg_sp):
    c = lax.axis_index("c")
    s = lax.axis_index("s")
    wid = c * NS + s
    pltpu.sync_copy(zeros_hbm, deg_sp.at[pl.ds(s * STRIPE, STRIPE)])
    pltpu.sync_copy(dst_hbm.at[wid], idx_v)
    pltpu.sync_copy(ones_hbm, ones_v)
    plsc.subcore_barrier()

    def body(j, carry):
        pltpu.sync_copy(ones_v, deg_sp.at[idx_v.at[j]], add=True)
        return carry

    lax.fori_loop(0, NITER, body, 0)
    plsc.subcore_barrier()
    pltpu.sync_copy(deg_sp.at[pl.ds(s * STRIPE, STRIPE)],
                    degp_out.at[c, pl.ds(s * STRIPE, STRIPE)])


_sc_degree = functools.partial(
    pl.kernel,
    out_type=jax.ShapeDtypeStruct((NC, NPAD, D), jnp.float32),
    mesh=_mesh,
    scratch_types=[
        pltpu.VMEM((NITER, K), jnp.int32),
        pltpu.VMEM((K, D), jnp.float32),
        pltpu.VMEM_SHARED((NPAD, D), jnp.float32),
    ],
)(_sc_degree_body)


NBUF = 3


def _sc_aggregate_body(hs_hbm, src_hbm, dst_hbm, zeros_hbm, aggp_out,
                       src_v, dst_v, rows_v, gsem, agg_sp):
    c = lax.axis_index("c")
    s = lax.axis_index("s")
    wid = c * NS + s

    def run_phase(base, cnt, zero_and_barrier):
        pltpu.sync_copy(src_hbm.at[wid, pl.ds(base, cnt)],
                        src_v.at[pl.ds(0, cnt)])
        pltpu.sync_copy(dst_hbm.at[wid, pl.ds(base, cnt)],
                        dst_v.at[pl.ds(0, cnt)])
        pltpu.async_copy(hs_hbm.at[src_v.at[0]], rows_v.at[0], gsem.at[0])
        pltpu.async_copy(hs_hbm.at[src_v.at[1]], rows_v.at[1], gsem.at[1])
        if zero_and_barrier:
            pltpu.sync_copy(zeros_hbm, agg_sp.at[pl.ds(s * STRIPE, STRIPE)])
            plsc.subcore_barrier()

        def body(t, carry):
            b = lax.rem(t, NBUF)
            nb = lax.rem(t + 2, NBUF)

            @pl.when(t + 2 < cnt)
            def _prefetch():
                pltpu.async_copy(hs_hbm.at[src_v.at[t + 2]], rows_v.at[nb],
                                 gsem.at[nb])

            pltpu.make_async_copy(hs_hbm.at[src_v.at[t]], rows_v.at[b],
                                  gsem.at[b]).wait()
            pltpu.sync_copy(rows_v.at[b], agg_sp.at[dst_v.at[t]], add=True)
            return carry

        lax.fori_loop(0, cnt, body, 0)

    run_phase(0, PH, True)
    run_phase(PH, PH, False)
    plsc.subcore_barrier()
    pltpu.sync_copy(agg_sp.at[pl.ds(s * STRIPE, STRIPE)],
                    aggp_out.at[c, pl.ds(s * STRIPE, STRIPE)])


_sc_aggregate = functools.partial(
    pl.kernel,
    out_type=jax.ShapeDtypeStruct((NC, NPAD, D), jnp.float32),
    mesh=_mesh,
    scratch_types=[
        pltpu.VMEM((PH, K), jnp.int32),
        pltpu.VMEM((PH, K), jnp.int32),
        pltpu.VMEM((NBUF, K, D), jnp.float32),
        pltpu.SemaphoreType.DMA((NBUF,)),
        pltpu.VMEM_SHARED((NPAD, D), jnp.float32),
    ],
)(_sc_aggregate_body)


def _mm_scale_body(x_ref, w_ref, degp_ref, o_ref):
    h = jnp.dot(x_ref[...], w_ref[...], preferred_element_type=jnp.float32)
    deg = 1.0 + degp_ref[0] + degp_ref[1]
    o_ref[...] = h * lax.rsqrt(deg)


def _tc_matmul_scale(x, W, degp):
    return pl.pallas_call(
        _mm_scale_body,
        grid=(NB,),
        in_specs=[
            pl.BlockSpec((RB, D), lambda i: (i, 0)),
            pl.BlockSpec((D, D), lambda i: (0, 0)),
            pl.BlockSpec((NC, RB, D), lambda i: (0, i, 0)),
        ],
        out_specs=pl.BlockSpec((RB, D), lambda i: (i, 0)),
        out_shape=jax.ShapeDtypeStruct((N, D), jnp.float32),
    )(x, W, degp)


def _final_body(aggp_ref, hs_ref, degp_ref, b_ref, lnw_ref, lnb_ref, a_ref,
                o_ref, t_vmem, acc):
    p = pl.program_id(0)
    i = pl.program_id(1)

    @pl.when(p == 0)
    def _phase0():
        @pl.when(i == 0)
        def _init():
            acc[0] = 0.0
            acc[1] = 0.0

        deg = 1.0 + degp_ref[0] + degp_ref[1]
        ag = aggp_ref[0] + aggp_ref[1] + hs_ref[...]
        t = ag * lax.rsqrt(deg) + b_ref[...]
        t_vmem[pl.ds(i * RB, RB), :] = t
        acc[0] += jnp.sum(t)
        acc[1] += jnp.sum(t * t)

    @pl.when(p == 1)
    def _phase1():
        inv_n = 1.0 / (N * D)
        m = acc[0] * inv_n
        var = acc[1] * inv_n - m * m
        std = jnp.sqrt(var)
        t = t_vmem[pl.ds(i * RB, RB), :]
        o = (t - m) / (std + EPS) * lnw_ref[...] + lnb_ref[...]
        a = a_ref[0, 0]
        o_ref[...] = jnp.where(o >= 0.0, o, a * o)


def _tc_finalize(aggp, hs, degp, b, ln_w, ln_b, prelu_a):
    return pl.pallas_call(
        _final_body,
        grid=(2, NB),
        in_specs=[
            pl.BlockSpec((NC, RB, D),
                         lambda p, i: (0, jnp.where(p == 0, i, 0), 0)),
            pl.BlockSpec((RB, D), lambda p, i: (jnp.where(p == 0, i, 0), 0)),
            pl.BlockSpec((NC, RB, D),
                         lambda p, i: (0, jnp.where(p == 0, i, 0), 0)),
            pl.BlockSpec((1, D), lambda p, i: (0, 0)),
            pl.BlockSpec((1, D), lambda p, i: (0, 0)),
            pl.BlockSpec((1, D), lambda p, i: (0, 0)),
            pl.BlockSpec((1, 1), lambda p, i: (0, 0)),
        ],
        out_specs=pl.BlockSpec((RB, D),
                               lambda p, i: (jnp.where(p == 0, 0, i), 0)),
        out_shape=jax.ShapeDtypeStruct((N, D), jnp.float32),
        scratch_shapes=[
            pltpu.VMEM((N, D), jnp.float32),
            pltpu.SMEM((2,), jnp.float32),
        ],
    )(aggp, hs, degp, b, ln_w, ln_b, prelu_a)


def kernel(x, edge_index, W, b, ln_w, ln_b, prelu_a):
    pad = EPAD - E
    pad_ar = jnp.arange(pad, dtype=jnp.int32)
    src = jnp.concatenate(
        [edge_index[0], pad_ar % N]).reshape(NW, NITER, K)
    dst = jnp.concatenate(
        [edge_index[1], N + pad_ar % (NPAD - N)]).reshape(NW, NITER, K)

    onesd = jnp.ones((K, D), jnp.float32)
    zerosd = jnp.zeros((STRIPE, D), jnp.float32)

    degp = _sc_degree(dst, onesd, zerosd)
    hs = _tc_matmul_scale(x, W, degp)
    aggp = _sc_aggregate(hs, src, dst, zerosd)

    b2 = b.reshape(1, D)
    lnw2 = ln_w.reshape(1, D)
    lnb2 = ln_b.reshape(1, D)
    a2 = prelu_a.reshape(1, 1)
    return _tc_finalize(aggp, hs, degp, b2, lnw2, lnb2, a2)


# speedup vs baseline: 2.4620x; 1.0047x over previous
import functools

import jax
import jax.numpy as jnp
from jax import lax
from jax.experimental import pallas as pl
from jax.experimental.pallas import tpu as pltpu
from jax.experimental.pallas import tpu_sc as plsc

N = 10000
D = 128
E = 320000
NC = 2
NS = 16
NW = NC * NS
K = 64
NITER = 160
EPAD = NW * NITER * K
NPAD = 10240
STRIPE = NPAD // NS
PH = NITER // 2
NB = 10
RB = N // NB
EPS = 1e-5

_mesh = plsc.VectorSubcoreMesh(core_axis_name="c", subcore_axis_name="s",
                               num_cores=NC, num_subcores=NS)


def _sc_degree_body(dst_hbm, ones_hbm, zeros_hbm, degp_out, idx_v, ones_v,
                    deg_sp):
    c = lax.axis_index("c")
    s = lax.axis_index("s")
    wid = c * NS + s
    pltpu.sync_copy(zeros_hbm, deg_sp.at[pl.ds(s * STRIPE, STRIPE)])
    pltpu.sync_copy(dst_hbm.at[wid], idx_v)
    pltpu.sync_copy(ones_hbm, ones_v)
    plsc.subcore_barrier()

    def body(j, carry):
        pltpu.sync_copy(ones_v, deg_sp.at[idx_v.at[j]], add=True)
        return carry

    lax.fori_loop(0, NITER, body, 0)
    plsc.subcore_barrier()
    pltpu.sync_copy(deg_sp.at[pl.ds(s * STRIPE, STRIPE)],
                    degp_out.at[c, pl.ds(s * STRIPE, STRIPE)])


_sc_degree = functools.partial(
    pl.kernel,
    out_type=jax.ShapeDtypeStruct((NC, NPAD, D), jnp.float32),
    mesh=_mesh,
    scratch_types=[
        pltpu.VMEM((NITER, K), jnp.int32),
        pltpu.VMEM((K, D), jnp.float32),
        pltpu.VMEM_SHARED((NPAD, D), jnp.float32),
    ],
)(_sc_degree_body)


NBUF = 3


def _sc_aggregate_body(hs_hbm, src_hbm, dst_hbm, zeros_hbm, aggp_out,
                       src_v, dst_v, rows_v, gsem, agg_sp):
    c = lax.axis_index("c")
    s = lax.axis_index("s")
    wid = c * NS + s

    def run_phase(base, cnt, zero_and_barrier):
        pltpu.sync_copy(src_hbm.at[wid, pl.ds(base, cnt)],
                        src_v.at[pl.ds(0, cnt)])
        pltpu.sync_copy(dst_hbm.at[wid, pl.ds(base, cnt)],
                        dst_v.at[pl.ds(0, cnt)])
        pltpu.async_copy(hs_hbm.at[src_v.at[0]], rows_v.at[0], gsem.at[0])
        pltpu.async_copy(hs_hbm.at[src_v.at[1]], rows_v.at[1], gsem.at[1])
        if zero_and_barrier:
            @pl.when(c == 0)
            def _seed_hs():
                @pl.when(s < NS - 1)
                def _full():
                    pltpu.sync_copy(hs_hbm.at[pl.ds(s * STRIPE, STRIPE)],
                                    agg_sp.at[pl.ds(s * STRIPE, STRIPE)])

                @pl.when(s == NS - 1)
                def _tail():
                    tb = (NS - 1) * STRIPE
                    pltpu.sync_copy(hs_hbm.at[pl.ds(tb, N - tb)],
                                    agg_sp.at[pl.ds(tb, N - tb)])
                    pltpu.sync_copy(zeros_hbm.at[pl.ds(0, NPAD - N)],
                                    agg_sp.at[pl.ds(N, NPAD - N)])

            @pl.when(c != 0)
            def _seed_zero():
                pltpu.sync_copy(zeros_hbm,
                                agg_sp.at[pl.ds(s * STRIPE, STRIPE)])

            plsc.subcore_barrier()

        def body(t, carry):
            b = lax.rem(t, NBUF)
            nb = lax.rem(t + 2, NBUF)

            @pl.when(t + 2 < cnt)
            def _prefetch():
                pltpu.async_copy(hs_hbm.at[src_v.at[t + 2]], rows_v.at[nb],
                                 gsem.at[nb])

            pltpu.make_async_copy(hs_hbm.at[src_v.at[t]], rows_v.at[b],
                                  gsem.at[b]).wait()
            pltpu.sync_copy(rows_v.at[b], agg_sp.at[dst_v.at[t]], add=True)
            return carry

        lax.fori_loop(0, cnt, body, 0)

    run_phase(0, PH, True)
    run_phase(PH, PH, False)
    plsc.subcore_barrier()
    pltpu.sync_copy(agg_sp.at[pl.ds(s * STRIPE, STRIPE)],
                    aggp_out.at[c, pl.ds(s * STRIPE, STRIPE)])


_sc_aggregate = functools.partial(
    pl.kernel,
    out_type=jax.ShapeDtypeStruct((NC, NPAD, D), jnp.float32),
    mesh=_mesh,
    scratch_types=[
        pltpu.VMEM((PH, K), jnp.int32),
        pltpu.VMEM((PH, K), jnp.int32),
        pltpu.VMEM((NBUF, K, D), jnp.float32),
        pltpu.SemaphoreType.DMA((NBUF,)),
        pltpu.VMEM_SHARED((NPAD, D), jnp.float32),
    ],
)(_sc_aggregate_body)


def _mm_scale_body(x_ref, w_ref, degp_ref, o_ref):
    h = jnp.dot(x_ref[...], w_ref[...], preferred_element_type=jnp.float32)
    deg = 1.0 + degp_ref[0] + degp_ref[1]
    o_ref[...] = h * lax.rsqrt(deg)


def _tc_matmul_scale(x, W, degp):
    return pl.pallas_call(
        _mm_scale_body,
        grid=(NB,),
        in_specs=[
            pl.BlockSpec((RB, D), lambda i: (i, 0)),
            pl.BlockSpec((D, D), lambda i: (0, 0)),
            pl.BlockSpec((NC, RB, D), lambda i: (0, i, 0)),
        ],
        out_specs=pl.BlockSpec((RB, D), lambda i: (i, 0)),
        out_shape=jax.ShapeDtypeStruct((N, D), jnp.float32),
    )(x, W, degp)


def _final_body(aggp_ref, degp_ref, b_ref, lnw_ref, lnb_ref, a_ref,
                o_ref, t_vmem, acc):
    p = pl.program_id(0)
    i = pl.program_id(1)

    @pl.when(p == 0)
    def _phase0():
        @pl.when(i == 0)
        def _init():
            acc[0] = 0.0
            acc[1] = 0.0

        deg = 1.0 + degp_ref[0] + degp_ref[1]
        ag = aggp_ref[0] + aggp_ref[1]
        t = ag * lax.rsqrt(deg) + b_ref[...]
        t_vmem[pl.ds(i * RB, RB), :] = t
        acc[0] += jnp.sum(t)
        acc[1] += jnp.sum(t * t)

    @pl.when(p == 1)
    def _phase1():
        inv_n = 1.0 / (N * D)
        m = acc[0] * inv_n
        var = acc[1] * inv_n - m * m
        std = jnp.sqrt(var)
        t = t_vmem[pl.ds(i * RB, RB), :]
        o = (t - m) / (std + EPS) * lnw_ref[...] + lnb_ref[...]
        a = a_ref[0, 0]
        o_ref[...] = jnp.where(o >= 0.0, o, a * o)


def _tc_finalize(aggp, degp, b, ln_w, ln_b, prelu_a):
    return pl.pallas_call(
        _final_body,
        grid=(2, NB),
        in_specs=[
            pl.BlockSpec((NC, RB, D),
                         lambda p, i: (0, jnp.where(p == 0, i, 0), 0)),
            pl.BlockSpec((NC, RB, D),
                         lambda p, i: (0, jnp.where(p == 0, i, 0), 0)),
            pl.BlockSpec((1, D), lambda p, i: (0, 0)),
            pl.BlockSpec((1, D), lambda p, i: (0, 0)),
            pl.BlockSpec((1, D), lambda p, i: (0, 0)),
            pl.BlockSpec((1, 1), lambda p, i: (0, 0)),
        ],
        out_specs=pl.BlockSpec((RB, D),
                               lambda p, i: (jnp.where(p == 0, 0, i), 0)),
        out_shape=jax.ShapeDtypeStruct((N, D), jnp.float32),
        scratch_shapes=[
            pltpu.VMEM((N, D), jnp.float32),
            pltpu.SMEM((2,), jnp.float32),
        ],
    )(aggp, degp, b, ln_w, ln_b, prelu_a)


def kernel(x, edge_index, W, b, ln_w, ln_b, prelu_a):
    pad = EPAD - E
    pad_ar = jnp.arange(pad, dtype=jnp.int32)
    src = jnp.concatenate(
        [edge_index[0], pad_ar % N]).reshape(NW, NITER, K)
    dst = jnp.concatenate(
        [edge_index[1], N + pad_ar % (NPAD - N)]).reshape(NW, NITER, K)

    onesd = jnp.ones((K, D), jnp.float32)
    zerosd = jnp.zeros((STRIPE, D), jnp.float32)

    degp = _sc_degree(dst, onesd, zerosd)
    hs = _tc_matmul_scale(x, W, degp)
    aggp = _sc_aggregate(hs, src, dst, zerosd)

    b2 = b.reshape(1, D)
    lnw2 = ln_w.reshape(1, D)
    lnb2 = ln_b.reshape(1, D)
    a2 = prelu_a.reshape(1, 1)
    return _tc_finalize(aggp, degp, b2, lnw2, lnb2, a2)
